# Initial kernel scaffold; baseline (speedup 1.0000x reference)
import jax
import jax.numpy as jnp
from jax.experimental import pallas as pl


def kernel(input_ids, elapsed_time, event_type, product_action, hashed_url, price_bucket, number_of_category_hash, category_hash_first_level, category_hash_second_level, category_hash_third_level, description_vector, image_vector, hour, weekday, weekend, id_table, elapsed_table, event_table, pa_table, url_table, price_table, nch_table, c1_table, c2_table, c3_table, hour_table, weekday_table, weekend_table, pos_table, W_item, b_item, W_lin, b_lin, gamma, beta):
    raise NotImplementedError("write your pallas kernel here")


# speedup vs baseline: 1.3379x; 1.3379x over previous
import functools

import jax
import jax.numpy as jnp
from jax import lax
from jax.experimental import pallas as pl
from jax.experimental.pallas import tpu as pltpu
from jax.experimental.pallas import tpu_sc as plsc

B, L = 1024, 50
E, H = 64, 128
ROWS = B * L
NF = 13
DIN = NF * E + 100

NC, NS = 2, 16
NW = NC * NS
CHUNK = 256
NCHUNK = ROWS // CHUNK

BR = 800
NBLK = ROWS // BR


def _gather_body(*refs):
    idxs = refs[:NF]
    tables = refs[NF:2 * NF]
    out_hbm = refs[2 * NF]
    idx_v, rows_v, sem = refs[2 * NF + 1:]
    wid = lax.axis_index("s") * NC + lax.axis_index("c")
    n_j = jnp.where(wid < NCHUNK % NW, NCHUNK // NW + 1, NCHUNK // NW)
    for f in range(NF):
        @pl.loop(0, n_j)
        def _(j):
            base = pl.multiple_of((wid + NW * j) * CHUNK, CHUNK)
            pltpu.sync_copy(idxs[f].at[pl.ds(base, CHUNK)], idx_v)
            pltpu.async_copy(tables[f].at[idx_v], rows_v, sem).wait()
            pltpu.sync_copy(rows_v, out_hbm.at[f].at[pl.ds(base, CHUNK)])


_sc_gather = functools.partial(
    pl.kernel,
    mesh=plsc.VectorSubcoreMesh(core_axis_name="c", subcore_axis_name="s"),
    out_type=jax.ShapeDtypeStruct((NF, ROWS, E), jnp.float32),
    scratch_types=[
        pltpu.VMEM((CHUNK,), jnp.int32),
        pltpu.VMEM((CHUNK, E), jnp.float32),
        pltpu.SemaphoreType.DMA,
    ],
    compiler_params=pltpu.CompilerParams(use_tc_tiling_on_sc=False),
)(_gather_body)


def _prep_body(wi_ref, wl_ref, bi_ref, bl_ref, pos_ref, m_ref, pe_ref):
    wl0 = wl_ref[:, :H]
    c = jnp.dot(wl0, wi_ref[...], preferred_element_type=jnp.float32)
    m_ref[:, : NF * E - 7 * E + 100] = c
    m_ref[:, NF * E - 7 * E + 100:] = wl_ref[:, H:]
    beff = lax.dot_general(bi_ref[...], wl0, (((1,), (1,)), ((), ())),
                           preferred_element_type=jnp.float32)
    pe_ref[...] = pos_ref[...] + beff + bl_ref[...]


def _prep(w_item, w_lin, b_item, b_lin, pos_table):
    return pl.pallas_call(
        _prep_body,
        out_shape=(
            jax.ShapeDtypeStruct((H, DIN), jnp.float32),
            jax.ShapeDtypeStruct((L, H), jnp.float32),
        ),
    )(w_item, w_lin, b_item.reshape(1, H), b_lin.reshape(1, H), pos_table)


def _main_body(x_ref, desc_ref, img_ref, m_ref, pe_ref, g_ref, b_ref, o_ref):
    parts = [x_ref[f] for f in range(6)]
    parts.append(desc_ref[...])
    parts.append(img_ref[...])
    parts.extend(x_ref[f] for f in range(6, NF))
    x = jnp.concatenate(parts, axis=-1)
    y = lax.dot_general(
        x.astype(jnp.bfloat16), m_ref[...].astype(jnp.bfloat16),
        (((1,), (1,)), ((), ())), preferred_element_type=jnp.float32)
    y = (y.reshape(BR // L, L, H) + pe_ref[...][None]).reshape(BR, H)
    mean = jnp.mean(y, axis=-1, keepdims=True)
    yc = y - mean
    var = jnp.mean(yc * yc, axis=-1, keepdims=True)
    o_ref[...] = yc * lax.rsqrt(var + 1e-12) * g_ref[...] + b_ref[...]


def _main(gx, desc, img, m_t, pe, gamma, beta):
    return pl.pallas_call(
        _main_body,
        grid=(NBLK,),
        in_specs=[
            pl.BlockSpec((NF, BR, E), lambda i: (0, i, 0)),
            pl.BlockSpec((BR, 50), lambda i: (i, 0)),
            pl.BlockSpec((BR, 50), lambda i: (i, 0)),
            pl.BlockSpec((H, DIN), lambda i: (0, 0)),
            pl.BlockSpec((L, H), lambda i: (0, 0)),
            pl.BlockSpec((1, H), lambda i: (0, 0)),
            pl.BlockSpec((1, H), lambda i: (0, 0)),
        ],
        out_specs=pl.BlockSpec((BR, H), lambda i: (i, 0)),
        out_shape=jax.ShapeDtypeStruct((ROWS, H), jnp.float32),
        compiler_params=pltpu.CompilerParams(
            dimension_semantics=("parallel",)),
    )(gx, desc, img, m_t, pe, gamma, beta)


def kernel(input_ids, elapsed_time, event_type, product_action, hashed_url,
           price_bucket, number_of_category_hash, category_hash_first_level,
           category_hash_second_level, category_hash_third_level,
           description_vector, image_vector, hour, weekday, weekend,
           id_table, elapsed_table, event_table, pa_table, url_table,
           price_table, nch_table, c1_table, c2_table, c3_table, hour_table,
           weekday_table, weekend_table, pos_table, W_item, b_item, W_lin,
           b_lin, gamma, beta):
    gx = _sc_gather(
        input_ids.reshape(ROWS), price_bucket.reshape(ROWS),
        number_of_category_hash.reshape(ROWS),
        category_hash_first_level.reshape(ROWS),
        category_hash_second_level.reshape(ROWS),
        category_hash_third_level.reshape(ROWS),
        elapsed_time.reshape(ROWS), event_type.reshape(ROWS),
        product_action.reshape(ROWS), hashed_url.reshape(ROWS),
        hour.reshape(ROWS), weekday.reshape(ROWS), weekend.reshape(ROWS),
        id_table, price_table, nch_table, c1_table, c2_table, c3_table,
        elapsed_table, event_table, pa_table, url_table, hour_table,
        weekday_table, weekend_table)
    m_t, pe = _prep(W_item, W_lin, b_item, b_lin, pos_table)
    out = _main(gx, description_vector.reshape(ROWS, 50),
                image_vector.reshape(ROWS, 50), m_t, pe,
                gamma.reshape(1, H), beta.reshape(1, H))
    return out.reshape(B, L, H)
